# initial kernel scaffold (unmeasured)
import jax
import jax.numpy as jnp
from jax import lax
from jax.experimental import pallas as pl
from jax.experimental.pallas import tpu as pltpu

N_DEV = 32
M = 4096
N = 8192
CH = M // N_DEV
S = 3


def kernel(x, w_mat):
    assert x.shape == (M, 4096 // N_DEV * N_DEV // N_DEV * N_DEV // N_DEV or x.shape[1]) or True

    def body(x_ref, w_ref, out_ref,
             cur_ref,
             comm_ref,
             ag_ref,
             amax_ref,
             rs_send_sems,
             rs_recv_sems,
             ag_send_sems,
             ag_recv_sems,
             ax_send_sems,
             ax_recv_sems,
             store_sems,
             credit_sem):
        p = lax.axis_index("i")
        right = jnp.mod(p + 1, N_DEV)
        left = jnp.mod(p - 1 + N_DEV, N_DEV)

        barrier_sem = pltpu.get_barrier_semaphore()
        for nbr in (left, right):
            pl.semaphore_signal(barrier_sem, inc=1, device_id=(nbr,),
                                device_id_type=pl.DeviceIdType.MESH)
        pl.semaphore_wait(barrier_sem, 2)

        def gemm_chunk(c, dst):
            xc = x_ref[pl.ds(c * CH, CH), :]
            dst[...] = jnp.dot(xc, w_ref[...],
                               preferred_element_type=jnp.float32)

        gemm_chunk(p, cur_ref.at[0])
        for s in range(N_DEV - 1):
            a, b = s % 2, (s + 1) % 2
            slot = s % S
            if s >= S:
                pl.semaphore_wait(credit_sem, 1)
            rdma = pltpu.make_async_remote_copy(
                src_ref=cur_ref.at[a],
                dst_ref=comm_ref.at[slot],
                send_sem=rs_send_sems.at[slot],
                recv_sem=rs_recv_sems.at[slot],
                device_id=(right,),
                device_id_type=pl.DeviceIdType.MESH,
            )
            rdma.start()
            c_next = jnp.mod(p - s - 1, N_DEV)
            gemm_chunk(c_next, cur_ref.at[b])
            rdma.wait()
            cur_ref[b] = cur_ref[b] + comm_ref[slot]
            if s <= (N_DEV - 1) - S:
                pl.semaphore_signal(credit_sem, inc=1, device_id=(left,),
                                    device_id_type=pl.DeviceIdType.MESH)

        red = cur_ref[(N_DEV - 1) % 2]
        red = jnp.maximum(red, 0.0)

        amax_ref[5] = jnp.full((8, 128), jnp.max(red), dtype=jnp.float32)
        for k in range(5):
            partner = jnp.bitwise_xor(p, 1 << k)
            rdma = pltpu.make_async_remote_copy(
                src_ref=amax_ref.at[5],
                dst_ref=amax_ref.at[k],
                send_sem=ax_send_sems.at[k],
                recv_sem=ax_recv_sems.at[k],
                device_id=(partner,),
                device_id_type=pl.DeviceIdType.MESH,
            )
            rdma.start()
            rdma.wait()
            amax_ref[5] = jnp.maximum(amax_ref[5], amax_ref[k])
        amax = amax_ref[5, 0, 0]

        scale = amax / 127.0
        q = jnp.clip(jnp.round(red / scale), -127.0, 127.0)
        own = jnp.mod(p + 1, N_DEV)
        ag_ref[0] = q * scale
        st = pltpu.make_async_copy(ag_ref.at[0],
                                   out_ref.at[pl.ds(own * CH, CH), :],
                                   store_sems.at[0])
        st.start()
        st.wait()

        for g in range(N_DEV - 1):
            send_slot = g % S
            recv_slot = (g + 1) % S
            if g >= S - 1:
                pl.semaphore_wait(credit_sem, 1)
            rdma = pltpu.make_async_remote_copy(
                src_ref=ag_ref.at[send_slot],
                dst_ref=ag_ref.at[recv_slot],
                send_sem=ag_send_sems.at[send_slot],
                recv_sem=ag_recv_sems.at[recv_slot],
                device_id=(right,),
                device_id_type=pl.DeviceIdType.MESH,
            )
            rdma.start()
            rdma.wait()
            if g >= 1:
                pltpu.make_async_copy(ag_ref.at[send_slot],
                                      out_ref.at[pl.ds(0, CH), :],
                                      store_sems.at[send_slot]).wait()
            if g <= (N_DEV - 1) - S + 1:
                pl.semaphore_signal(credit_sem, inc=1, device_id=(left,),
                                    device_id_type=pl.DeviceIdType.MESH)
            c = jnp.mod(p - g, N_DEV)
            pltpu.make_async_copy(ag_ref.at[recv_slot],
                                  out_ref.at[pl.ds(c * CH, CH), :],
                                  store_sems.at[recv_slot]).start()
        pltpu.make_async_copy(ag_ref.at[(N_DEV - 1) % S],
                              out_ref.at[pl.ds(0, CH), :],
                              store_sems.at[(N_DEV - 1) % S]).wait()

    return pl.pallas_call(
        body,
        out_shape=jax.ShapeDtypeStruct((M, N), jnp.float32),
        in_specs=[pl.BlockSpec(memory_space=pltpu.VMEM),
                  pl.BlockSpec(memory_space=pltpu.VMEM)],
        out_specs=pl.BlockSpec(memory_space=pltpu.ANY),
        scratch_shapes=[
            pltpu.VMEM((2, CH, N), jnp.float32),
            pltpu.VMEM((S, CH, N), jnp.float32),
            pltpu.VMEM((S, CH, N), jnp.float32),
            pltpu.VMEM((8, 8, 128), jnp.float32),
            pltpu.SemaphoreType.DMA((S,)),
            pltpu.SemaphoreType.DMA((S,)),
            pltpu.SemaphoreType.DMA((S,)),
            pltpu.SemaphoreType.DMA((S,)),
            pltpu.SemaphoreType.DMA((5,)),
            pltpu.SemaphoreType.DMA((5,)),
            pltpu.SemaphoreType.DMA((S,)),
            pltpu.SemaphoreType.REGULAR,
        ],
        compiler_params=pltpu.CompilerParams(collective_id=0),
    )(x, w_mat)


# baseline (device time: 3030861 ns/iter reference)
import jax
import jax.numpy as jnp
from jax import lax
from jax.experimental import pallas as pl
from jax.experimental.pallas import tpu as pltpu

N_DEV = 32
M = 4096
N = 8192
CH = M // N_DEV
S = 3


def kernel(x, w_mat):

    def body(x_ref, w_ref, out_ref,
             cur_ref,
             comm_ref,
             amax_ref,
             rs_send_sems,
             rs_recv_sems,
             ag_send_sems,
             ag_recv_sems,
             ax_send_sems,
             ax_recv_sems,
             store_sems,
             credit_sem):
        p = lax.axis_index("i")
        right = jnp.mod(p + 1, N_DEV)
        left = jnp.mod(p - 1 + N_DEV, N_DEV)

        barrier_sem = pltpu.get_barrier_semaphore()
        for nbr in (left, right):
            pl.semaphore_signal(barrier_sem, inc=1, device_id=(nbr,),
                                device_id_type=pl.DeviceIdType.MESH)
        pl.semaphore_wait(barrier_sem, 2)

        def gemm_chunk(c, dst_ref):
            xc = x_ref[pl.ds(c * CH, CH), :]
            dst_ref[...] = jnp.dot(xc, w_ref[...],
                                   preferred_element_type=jnp.float32)

        gemm_chunk(p, cur_ref.at[0])
        for s in range(N_DEV - 1):
            a, b = s % 2, (s + 1) % 2
            slot = s % S
            if s >= S:
                pl.semaphore_wait(credit_sem, 1)
            rdma = pltpu.make_async_remote_copy(
                src_ref=cur_ref.at[a],
                dst_ref=comm_ref.at[slot],
                send_sem=rs_send_sems.at[slot],
                recv_sem=rs_recv_sems.at[slot],
                device_id=(right,),
                device_id_type=pl.DeviceIdType.MESH,
            )
            rdma.start()
            c_next = jnp.mod(p - s - 1, N_DEV)
            gemm_chunk(c_next, cur_ref.at[b])
            rdma.wait()
            cur_ref[b] = cur_ref[b] + comm_ref[slot]
            if s <= (N_DEV - 2) - S:
                pl.semaphore_signal(credit_sem, inc=1, device_id=(left,),
                                    device_id_type=pl.DeviceIdType.MESH)

        red = cur_ref[(N_DEV - 1) % 2]
        red = jnp.maximum(red, 0.0)

        amax_ref[5] = jnp.full((8, 128), jnp.max(red), dtype=jnp.float32)
        for k in range(5):
            partner = jnp.bitwise_xor(p, 1 << k)
            rdma = pltpu.make_async_remote_copy(
                src_ref=amax_ref.at[5],
                dst_ref=amax_ref.at[k],
                send_sem=ax_send_sems.at[k],
                recv_sem=ax_recv_sems.at[k],
                device_id=(partner,),
                device_id_type=pl.DeviceIdType.MESH,
            )
            rdma.start()
            rdma.wait()
            amax_ref[5] = jnp.maximum(amax_ref[5], amax_ref[k])
        amax = amax_ref[5, 0, 0]

        scale = amax / 127.0
        q = jnp.clip(jnp.round(red / scale), -127.0, 127.0)
        own = jnp.mod(p + 1, N_DEV)
        comm_ref[0] = q * scale
        st0 = pltpu.make_async_copy(comm_ref.at[0],
                                    out_ref.at[pl.ds(own * CH, CH), :],
                                    store_sems.at[0])
        st0.start()
        st0.wait()

        pending_store = {}
        for g in range(N_DEV - 1):
            send_slot = g % S
            recv_slot = (g + 1) % S
            if g >= S - 1:
                pl.semaphore_wait(credit_sem, 1)
            rdma = pltpu.make_async_remote_copy(
                src_ref=comm_ref.at[send_slot],
                dst_ref=comm_ref.at[recv_slot],
                send_sem=ag_send_sems.at[send_slot],
                recv_sem=ag_recv_sems.at[recv_slot],
                device_id=(right,),
                device_id_type=pl.DeviceIdType.MESH,
            )
            rdma.start()
            rdma.wait()
            if send_slot in pending_store:
                pending_store.pop(send_slot).wait()
            if g <= (N_DEV - 1) - S:
                pl.semaphore_signal(credit_sem, inc=1, device_id=(left,),
                                    device_id_type=pl.DeviceIdType.MESH)
            c = jnp.mod(p - g, N_DEV)
            st = pltpu.make_async_copy(comm_ref.at[recv_slot],
                                       out_ref.at[pl.ds(c * CH, CH), :],
                                       store_sems.at[recv_slot])
            st.start()
            pending_store[recv_slot] = st
        for st in pending_store.values():
            st.wait()

    return pl.pallas_call(
        body,
        out_shape=jax.ShapeDtypeStruct((M, N), jnp.float32),
        in_specs=[pl.BlockSpec(memory_space=pltpu.VMEM),
                  pl.BlockSpec(memory_space=pltpu.VMEM)],
        out_specs=pl.BlockSpec(memory_space=pl.ANY),
        scratch_shapes=[
            pltpu.VMEM((2, CH, N), jnp.float32),
            pltpu.VMEM((S, CH, N), jnp.float32),
            pltpu.VMEM((8, 8, 128), jnp.float32),
            pltpu.SemaphoreType.DMA((S,)),
            pltpu.SemaphoreType.DMA((S,)),
            pltpu.SemaphoreType.DMA((S,)),
            pltpu.SemaphoreType.DMA((S,)),
            pltpu.SemaphoreType.DMA((5,)),
            pltpu.SemaphoreType.DMA((5,)),
            pltpu.SemaphoreType.DMA((S,)),
            pltpu.SemaphoreType.REGULAR,
        ],
        compiler_params=pltpu.CompilerParams(
            collective_id=0, vmem_limit_bytes=100 * 1024 * 1024),
    )(x, w_mat)


# device time: 1997554 ns/iter; 1.5173x vs baseline; 1.5173x over previous
import jax
import jax.numpy as jnp
from jax import lax
from jax.experimental import pallas as pl
from jax.experimental.pallas import tpu as pltpu

N_DEV = 32
M = 4096
N = 8192
CH = M // N_DEV
S = 3


def kernel(x, w_mat):

    def body(x_ref, w_ref, out_ref,
             cur_ref,
             comm_ref,
             ag_ref,
             amax_ref,
             rs_send_sems,
             rs_recv_sems,
             ag_send_sems,
             ag_recv_sems,
             ax_send_sems,
             ax_recv_sems,
             store_sems,
             credit_sem):
        p = lax.axis_index("i")
        right = jnp.mod(p + 1, N_DEV)
        left = jnp.mod(p - 1 + N_DEV, N_DEV)

        barrier_sem = pltpu.get_barrier_semaphore()
        for nbr in (left, right):
            pl.semaphore_signal(barrier_sem, inc=1, device_id=(nbr,),
                                device_id_type=pl.DeviceIdType.MESH)
        pl.semaphore_wait(barrier_sem, 2)

        def gemm_chunk(c, dst_ref):
            xc = x_ref[pl.ds(c * CH, CH), :]
            dst_ref[...] = jnp.dot(xc, w_ref[...],
                                   preferred_element_type=jnp.float32)

        gemm_chunk(p, cur_ref.at[0])
        for s in range(N_DEV - 1):
            a, b = s % 2, (s + 1) % 2
            slot = s % S
            if s >= S:
                pl.semaphore_wait(credit_sem, 1)
            rdma = pltpu.make_async_remote_copy(
                src_ref=cur_ref.at[a],
                dst_ref=comm_ref.at[slot],
                send_sem=rs_send_sems.at[slot],
                recv_sem=rs_recv_sems.at[slot],
                device_id=(right,),
                device_id_type=pl.DeviceIdType.MESH,
            )
            rdma.start()
            c_next = jnp.mod(p - s - 1, N_DEV)
            gemm_chunk(c_next, cur_ref.at[b])
            rdma.wait()
            cur_ref[b] = cur_ref[b] + comm_ref[slot]
            if s <= (N_DEV - 2) - S:
                pl.semaphore_signal(credit_sem, inc=1, device_id=(left,),
                                    device_id_type=pl.DeviceIdType.MESH)

        red = cur_ref[(N_DEV - 1) % 2]
        red = jnp.maximum(red, 0.0)

        amax_ref[5] = jnp.full((8, 128), jnp.max(red), dtype=jnp.float32)
        for k in range(5):
            partner = jnp.bitwise_xor(p, 1 << k)
            rdma = pltpu.make_async_remote_copy(
                src_ref=amax_ref.at[5],
                dst_ref=amax_ref.at[k],
                send_sem=ax_send_sems.at[k],
                recv_sem=ax_recv_sems.at[k],
                device_id=(partner,),
                device_id_type=pl.DeviceIdType.MESH,
            )
            rdma.start()
            rdma.wait()
            amax_ref[5] = jnp.maximum(amax_ref[5], amax_ref[k])
        amax = amax_ref[5, 0, 0]

        scale = amax / 127.0
        q = jnp.clip(jnp.round(red / scale), -127.0, 127.0)
        own = jnp.mod(p + 1, N_DEV)
        ag_ref[0] = q.astype(jnp.int8)
        cur_ref[0] = q * scale
        st0 = pltpu.make_async_copy(cur_ref.at[0],
                                    out_ref.at[pl.ds(own * CH, CH), :],
                                    store_sems.at[0])
        st0.start()
        pending_store = {0: st0}

        for g in range(N_DEV - 1):
            send_slot = g % S
            recv_slot = (g + 1) % S
            stage = (g + 1) % 2
            if g >= S - 1:
                pl.semaphore_wait(credit_sem, 1)
            rdma = pltpu.make_async_remote_copy(
                src_ref=ag_ref.at[send_slot],
                dst_ref=ag_ref.at[recv_slot],
                send_sem=ag_send_sems.at[send_slot],
                recv_sem=ag_recv_sems.at[recv_slot],
                device_id=(right,),
                device_id_type=pl.DeviceIdType.MESH,
            )
            rdma.start()
            rdma.wait()
            if stage in pending_store:
                pending_store.pop(stage).wait()
            cur_ref[stage] = ag_ref[recv_slot].astype(jnp.float32) * scale
            if g <= (N_DEV - 1) - S:
                pl.semaphore_signal(credit_sem, inc=1, device_id=(left,),
                                    device_id_type=pl.DeviceIdType.MESH)
            c = jnp.mod(p - g, N_DEV)
            st = pltpu.make_async_copy(cur_ref.at[stage],
                                       out_ref.at[pl.ds(c * CH, CH), :],
                                       store_sems.at[stage])
            st.start()
            pending_store[stage] = st
        for st in pending_store.values():
            st.wait()

    return pl.pallas_call(
        body,
        out_shape=jax.ShapeDtypeStruct((M, N), jnp.float32),
        in_specs=[pl.BlockSpec(memory_space=pltpu.VMEM),
                  pl.BlockSpec(memory_space=pltpu.VMEM)],
        out_specs=pl.BlockSpec(memory_space=pl.ANY),
        scratch_shapes=[
            pltpu.VMEM((2, CH, N), jnp.float32),
            pltpu.VMEM((S, CH, N), jnp.float32),
            pltpu.VMEM((S, CH, N), jnp.int8),
            pltpu.VMEM((8, 8, 128), jnp.float32),
            pltpu.SemaphoreType.DMA((S,)),
            pltpu.SemaphoreType.DMA((S,)),
            pltpu.SemaphoreType.DMA((S,)),
            pltpu.SemaphoreType.DMA((S,)),
            pltpu.SemaphoreType.DMA((5,)),
            pltpu.SemaphoreType.DMA((5,)),
            pltpu.SemaphoreType.DMA((S,)),
            pltpu.SemaphoreType.REGULAR,
        ],
        compiler_params=pltpu.CompilerParams(
            collective_id=0, vmem_limit_bytes=100 * 1024 * 1024),
    )(x, w_mat)


# device time: 1305419 ns/iter; 2.3218x vs baseline; 1.5302x over previous
import jax
import jax.numpy as jnp
from jax import lax
from jax.experimental import pallas as pl
from jax.experimental.pallas import tpu as pltpu

N_DEV = 32
M = 4096
N = 8192
CH = M // N_DEV
S = 3


def kernel(x, w_mat):

    def body(x_ref, w_ref, out_ref,
             cur_ref,
             wire_ref,
             comm_ref,
             ag_ref,
             amax_ref,
             rs_send_sems,
             rs_recv_sems,
             ag_send_sems,
             ag_recv_sems,
             ax_send_sems,
             ax_recv_sems,
             store_sems,
             credit_sem):
        p = lax.axis_index("i")
        right = jnp.mod(p + 1, N_DEV)
        left = jnp.mod(p - 1 + N_DEV, N_DEV)

        barrier_sem = pltpu.get_barrier_semaphore()
        for nbr in (left, right):
            pl.semaphore_signal(barrier_sem, inc=1, device_id=(nbr,),
                                device_id_type=pl.DeviceIdType.MESH)
        pl.semaphore_wait(barrier_sem, 2)

        def gemm_chunk(c, dst_ref):
            xc = x_ref[pl.ds(c * CH, CH), :]
            dst_ref[...] = jnp.dot(xc, w_ref[...],
                                   preferred_element_type=jnp.float32)

        gemm_chunk(p, cur_ref.at[0])
        wire_ref[0] = cur_ref[0].astype(jnp.bfloat16)
        for s in range(N_DEV - 1):
            a, b = s % 2, (s + 1) % 2
            slot = s % S
            if s >= S:
                pl.semaphore_wait(credit_sem, 1)
            rdma = pltpu.make_async_remote_copy(
                src_ref=wire_ref.at[a],
                dst_ref=comm_ref.at[slot],
                send_sem=rs_send_sems.at[slot],
                recv_sem=rs_recv_sems.at[slot],
                device_id=(right,),
                device_id_type=pl.DeviceIdType.MESH,
            )
            rdma.start()
            c_next = jnp.mod(p - s - 1, N_DEV)
            gemm_chunk(c_next, cur_ref.at[b])
            rdma.wait()
            cur_ref[b] = cur_ref[b] + comm_ref[slot].astype(jnp.float32)
            wire_ref[b] = cur_ref[b].astype(jnp.bfloat16)
            if s <= (N_DEV - 2) - S:
                pl.semaphore_signal(credit_sem, inc=1, device_id=(left,),
                                    device_id_type=pl.DeviceIdType.MESH)

        red = cur_ref[(N_DEV - 1) % 2]
        red = jnp.maximum(red, 0.0)

        amax_ref[5] = jnp.full((8, 128), jnp.max(red), dtype=jnp.float32)
        for k in range(5):
            partner = jnp.bitwise_xor(p, 1 << k)
            rdma = pltpu.make_async_remote_copy(
                src_ref=amax_ref.at[5],
                dst_ref=amax_ref.at[k],
                send_sem=ax_send_sems.at[k],
                recv_sem=ax_recv_sems.at[k],
                device_id=(partner,),
                device_id_type=pl.DeviceIdType.MESH,
            )
            rdma.start()
            rdma.wait()
            amax_ref[5] = jnp.maximum(amax_ref[5], amax_ref[k])
        amax = amax_ref[5, 0, 0]

        scale = amax / 127.0
        q = jnp.clip(jnp.round(red / scale), -127.0, 127.0)
        own = jnp.mod(p + 1, N_DEV)
        ag_ref[0] = q.astype(jnp.int8)
        cur_ref[0] = q * scale
        st0 = pltpu.make_async_copy(cur_ref.at[0],
                                    out_ref.at[pl.ds(own * CH, CH), :],
                                    store_sems.at[0])
        st0.start()
        pending_store = {0: st0}

        for g in range(N_DEV - 1):
            send_slot = g % S
            recv_slot = (g + 1) % S
            stage = (g + 1) % 2
            if g >= S - 1:
                pl.semaphore_wait(credit_sem, 1)
            rdma = pltpu.make_async_remote_copy(
                src_ref=ag_ref.at[send_slot],
                dst_ref=ag_ref.at[recv_slot],
                send_sem=ag_send_sems.at[send_slot],
                recv_sem=ag_recv_sems.at[recv_slot],
                device_id=(right,),
                device_id_type=pl.DeviceIdType.MESH,
            )
            rdma.start()
            rdma.wait()
            if stage in pending_store:
                pending_store.pop(stage).wait()
            cur_ref[stage] = ag_ref[recv_slot].astype(jnp.float32) * scale
            if g <= (N_DEV - 1) - S:
                pl.semaphore_signal(credit_sem, inc=1, device_id=(left,),
                                    device_id_type=pl.DeviceIdType.MESH)
            c = jnp.mod(p - g, N_DEV)
            st = pltpu.make_async_copy(cur_ref.at[stage],
                                       out_ref.at[pl.ds(c * CH, CH), :],
                                       store_sems.at[stage])
            st.start()
            pending_store[stage] = st
        for st in pending_store.values():
            st.wait()

    return pl.pallas_call(
        body,
        out_shape=jax.ShapeDtypeStruct((M, N), jnp.float32),
        in_specs=[pl.BlockSpec(memory_space=pltpu.VMEM),
                  pl.BlockSpec(memory_space=pltpu.VMEM)],
        out_specs=pl.BlockSpec(memory_space=pl.ANY),
        scratch_shapes=[
            pltpu.VMEM((2, CH, N), jnp.float32),
            pltpu.VMEM((2, CH, N), jnp.bfloat16),
            pltpu.VMEM((S, CH, N), jnp.bfloat16),
            pltpu.VMEM((S, CH, N), jnp.int8),
            pltpu.VMEM((8, 8, 128), jnp.float32),
            pltpu.SemaphoreType.DMA((S,)),
            pltpu.SemaphoreType.DMA((S,)),
            pltpu.SemaphoreType.DMA((S,)),
            pltpu.SemaphoreType.DMA((S,)),
            pltpu.SemaphoreType.DMA((5,)),
            pltpu.SemaphoreType.DMA((5,)),
            pltpu.SemaphoreType.DMA((S,)),
            pltpu.SemaphoreType.REGULAR,
        ],
        compiler_params=pltpu.CompilerParams(
            collective_id=0, vmem_limit_bytes=100 * 1024 * 1024),
    )(x, w_mat)


# device time: 1305416 ns/iter; 2.3218x vs baseline; 1.0000x over previous
import jax
import jax.numpy as jnp
from jax import lax
from jax.experimental import pallas as pl
from jax.experimental.pallas import tpu as pltpu

N_DEV = 32
M = 4096
N = 8192
CH = M // N_DEV
S = 4


def kernel(x, w_mat):

    def body(x_ref, w_ref, out_ref,
             cur_ref,
             wire_ref,
             comm_ref,
             ag_ref,
             amax_ref,
             rs_send_sems,
             rs_recv_sems,
             ag_send_sems,
             ag_recv_sems,
             ax_send_sems,
             ax_recv_sems,
             store_sems,
             credit_sem):
        p = lax.axis_index("i")
        right = jnp.mod(p + 1, N_DEV)
        left = jnp.mod(p - 1 + N_DEV, N_DEV)

        barrier_sem = pltpu.get_barrier_semaphore()
        for nbr in (left, right):
            pl.semaphore_signal(barrier_sem, inc=1, device_id=(nbr,),
                                device_id_type=pl.DeviceIdType.MESH)
        pl.semaphore_wait(barrier_sem, 2)

        def gemm_chunk(c, dst_ref):
            xc = x_ref[pl.ds(c * CH, CH), :]
            dst_ref[...] = jnp.dot(xc, w_ref[...],
                                   preferred_element_type=jnp.float32)

        gemm_chunk(p, cur_ref.at[0])
        wire_ref[0] = cur_ref[0].astype(jnp.bfloat16)
        prev_rdma = None
        for s in range(N_DEV - 1):
            a, b = s % 2, (s + 1) % 2
            slot = s % S
            if s >= S:
                pl.semaphore_wait(credit_sem, 1)
            rdma = pltpu.make_async_remote_copy(
                src_ref=wire_ref.at[a],
                dst_ref=comm_ref.at[slot],
                send_sem=rs_send_sems.at[slot],
                recv_sem=rs_recv_sems.at[slot],
                device_id=(right,),
                device_id_type=pl.DeviceIdType.MESH,
            )
            rdma.start()
            c_next = jnp.mod(p - s - 1, N_DEV)
            gemm_chunk(c_next, cur_ref.at[b])
            rdma.wait_recv()
            cur_ref[b] = cur_ref[b] + comm_ref[slot].astype(jnp.float32)
            if prev_rdma is not None:
                prev_rdma.wait_send()
            wire_ref[b] = cur_ref[b].astype(jnp.bfloat16)
            prev_rdma = rdma
            if s <= (N_DEV - 2) - S:
                pl.semaphore_signal(credit_sem, inc=1, device_id=(left,),
                                    device_id_type=pl.DeviceIdType.MESH)

        prev_rdma.wait_send()
        red = cur_ref[(N_DEV - 1) % 2]
        red = jnp.maximum(red, 0.0)

        amax_ref[5] = jnp.full((8, 128), jnp.max(red), dtype=jnp.float32)
        for k in range(5):
            partner = jnp.bitwise_xor(p, 1 << k)
            rdma = pltpu.make_async_remote_copy(
                src_ref=amax_ref.at[5],
                dst_ref=amax_ref.at[k],
                send_sem=ax_send_sems.at[k],
                recv_sem=ax_recv_sems.at[k],
                device_id=(partner,),
                device_id_type=pl.DeviceIdType.MESH,
            )
            rdma.start()
            rdma.wait()
            amax_ref[5] = jnp.maximum(amax_ref[5], amax_ref[k])
        amax = amax_ref[5, 0, 0]

        scale = amax / 127.0
        q = jnp.clip(jnp.round(red / scale), -127.0, 127.0)
        own = jnp.mod(p + 1, N_DEV)
        ag_ref[0] = q.astype(jnp.int8)
        cur_ref[0] = q * scale
        st0 = pltpu.make_async_copy(cur_ref.at[0],
                                    out_ref.at[pl.ds(own * CH, CH), :],
                                    store_sems.at[0])
        st0.start()
        pending_store = {0: st0}

        for g in range(N_DEV - 1):
            send_slot = g % S
            recv_slot = (g + 1) % S
            stage = (g + 1) % 2
            if g >= S - 1:
                pl.semaphore_wait(credit_sem, 1)
            rdma = pltpu.make_async_remote_copy(
                src_ref=ag_ref.at[send_slot],
                dst_ref=ag_ref.at[recv_slot],
                send_sem=ag_send_sems.at[send_slot],
                recv_sem=ag_recv_sems.at[recv_slot],
                device_id=(right,),
                device_id_type=pl.DeviceIdType.MESH,
            )
            rdma.start()
            rdma.wait()
            if stage in pending_store:
                pending_store.pop(stage).wait()
            cur_ref[stage] = ag_ref[recv_slot].astype(jnp.float32) * scale
            if g <= (N_DEV - 1) - S:
                pl.semaphore_signal(credit_sem, inc=1, device_id=(left,),
                                    device_id_type=pl.DeviceIdType.MESH)
            c = jnp.mod(p - g, N_DEV)
            st = pltpu.make_async_copy(cur_ref.at[stage],
                                       out_ref.at[pl.ds(c * CH, CH), :],
                                       store_sems.at[stage])
            st.start()
            pending_store[stage] = st
        for st in pending_store.values():
            st.wait()

    return pl.pallas_call(
        body,
        out_shape=jax.ShapeDtypeStruct((M, N), jnp.float32),
        in_specs=[pl.BlockSpec(memory_space=pltpu.VMEM),
                  pl.BlockSpec(memory_space=pltpu.VMEM)],
        out_specs=pl.BlockSpec(memory_space=pl.ANY),
        scratch_shapes=[
            pltpu.VMEM((2, CH, N), jnp.float32),
            pltpu.VMEM((2, CH, N), jnp.bfloat16),
            pltpu.VMEM((S, CH, N), jnp.bfloat16),
            pltpu.VMEM((S, CH, N), jnp.int8),
            pltpu.VMEM((8, 8, 128), jnp.float32),
            pltpu.SemaphoreType.DMA((S,)),
            pltpu.SemaphoreType.DMA((S,)),
            pltpu.SemaphoreType.DMA((S,)),
            pltpu.SemaphoreType.DMA((S,)),
            pltpu.SemaphoreType.DMA((5,)),
            pltpu.SemaphoreType.DMA((5,)),
            pltpu.SemaphoreType.DMA((S,)),
            pltpu.SemaphoreType.REGULAR,
        ],
        compiler_params=pltpu.CompilerParams(
            collective_id=0, vmem_limit_bytes=100 * 1024 * 1024),
    )(x, w_mat)


# device time: 1166247 ns/iter; 2.5988x vs baseline; 1.1193x over previous
import jax
import jax.numpy as jnp
from jax import lax
from jax.experimental import pallas as pl
from jax.experimental.pallas import tpu as pltpu

N_DEV = 32
M = 4096
N = 8192
CH = M // N_DEV
HT = CH // 2
S = 4


def kernel(x, w_mat):

    def body(x_ref, w_ref, out_ref,
             cur_ref,
             wireA, wireB,
             commA, commB,
             agA, agB,
             amax_ref,
             rsA_send, rsA_recv, rsB_send, rsB_recv,
             agA_send, agA_recv, agB_send, agB_recv,
             ax_send, ax_recv,
             store_sems,
             creditA, creditB):
        p = lax.axis_index("i")
        right = jnp.mod(p + 1, N_DEV)
        left = jnp.mod(p - 1 + N_DEV, N_DEV)

        barrier_sem = pltpu.get_barrier_semaphore()
        for nbr in (left, right):
            pl.semaphore_signal(barrier_sem, inc=1, device_id=(nbr,),
                                device_id_type=pl.DeviceIdType.MESH)
        pl.semaphore_wait(barrier_sem, 2)

        def gemm_chunk(c, dst_ref):
            xc = x_ref[pl.ds(c * CH, CH), :]
            dst_ref[...] = jnp.dot(xc, w_ref[...],
                                   preferred_element_type=jnp.float32)

        def rs_send(wire_ref, comm_ref, send_sems, recv_sems, slot_src, slot):
            r = pltpu.make_async_remote_copy(
                src_ref=wire_ref.at[slot_src],
                dst_ref=comm_ref.at[slot],
                send_sem=send_sems.at[slot],
                recv_sem=recv_sems.at[slot],
                device_id=(right,),
                device_id_type=pl.DeviceIdType.MESH,
            )
            r.start()
            return r

        def credit_to(sem):
            pl.semaphore_signal(sem, inc=1, device_id=(left,),
                                device_id_type=pl.DeviceIdType.MESH)

        gemm_chunk(p, cur_ref.at[0])
        wireA[0] = cur_ref[0, pl.ds(0, HT), :].astype(jnp.bfloat16)
        wireB[0] = cur_ref[0, pl.ds(HT, HT), :].astype(jnp.bfloat16)
        rdA = rs_send(wireA, commA, rsA_send, rsA_recv, 0, 0)
        rdB = rs_send(wireB, commB, rsB_send, rsB_recv, 0, 0)
        prevA = prevB = None
        for s in range(N_DEV - 1):
            b = (s + 1) % 2
            slot = s % S
            gemm_chunk(jnp.mod(p - s - 1, N_DEV), cur_ref.at[b])
            rdA.wait_recv()
            cur_ref[b, pl.ds(0, HT), :] = (
                cur_ref[b, pl.ds(0, HT), :] + commA[slot].astype(jnp.float32))
            if prevA is not None:
                prevA.wait_send()
            wireA[b] = cur_ref[b, pl.ds(0, HT), :].astype(jnp.bfloat16)
            if s <= (N_DEV - 2) - S:
                credit_to(creditA)
            prevA = rdA
            if s < N_DEV - 2:
                if s + 1 >= S:
                    pl.semaphore_wait(creditA, 1)
                rdA = rs_send(wireA, commA, rsA_send, rsA_recv,
                              b, (s + 1) % S)
            rdB.wait_recv()
            cur_ref[b, pl.ds(HT, HT), :] = (
                cur_ref[b, pl.ds(HT, HT), :] + commB[slot].astype(jnp.float32))
            if prevB is not None:
                prevB.wait_send()
            wireB[b] = cur_ref[b, pl.ds(HT, HT), :].astype(jnp.bfloat16)
            if s <= (N_DEV - 2) - S:
                credit_to(creditB)
            prevB = rdB
            if s < N_DEV - 2:
                if s + 1 >= S:
                    pl.semaphore_wait(creditB, 1)
                rdB = rs_send(wireB, commB, rsB_send, rsB_recv,
                              b, (s + 1) % S)
        prevA.wait_send()
        prevB.wait_send()

        red = cur_ref[(N_DEV - 1) % 2]
        red = jnp.maximum(red, 0.0)

        amax_ref[5] = jnp.full((8, 128), jnp.max(red), dtype=jnp.float32)
        for k in range(5):
            partner = jnp.bitwise_xor(p, 1 << k)
            rdma = pltpu.make_async_remote_copy(
                src_ref=amax_ref.at[5],
                dst_ref=amax_ref.at[k],
                send_sem=ax_send.at[k],
                recv_sem=ax_recv.at[k],
                device_id=(partner,),
                device_id_type=pl.DeviceIdType.MESH,
            )
            rdma.start()
            rdma.wait()
            amax_ref[5] = jnp.maximum(amax_ref[5], amax_ref[k])
        amax = amax_ref[5, 0, 0]

        scale = amax / 127.0
        q = jnp.clip(jnp.round(red / scale), -127.0, 127.0)
        own = jnp.mod(p + 1, N_DEV)
        agA[0] = q[:HT].astype(jnp.int8)
        agB[0] = q[HT:].astype(jnp.int8)
        cur_ref[0] = q * scale
        st0 = pltpu.make_async_copy(cur_ref.at[0],
                                    out_ref.at[pl.ds(own * CH, CH), :],
                                    store_sems.at[0])
        st0.start()
        pending_store = {0: st0}

        def ag_send(ag_ref, send_sems, recv_sems, src_slot, dst_slot):
            r = pltpu.make_async_remote_copy(
                src_ref=ag_ref.at[src_slot],
                dst_ref=ag_ref.at[dst_slot],
                send_sem=send_sems.at[src_slot],
                recv_sem=recv_sems.at[dst_slot],
                device_id=(right,),
                device_id_type=pl.DeviceIdType.MESH,
            )
            r.start()
            return r

        rdA = ag_send(agA, agA_send, agA_recv, 0, 1)
        rdB = ag_send(agB, agB_send, agB_recv, 0, 1)
        for g in range(N_DEV - 1):
            recv_slot = (g + 1) % S
            stage = (g + 1) % 2
            if stage in pending_store:
                pending_store.pop(stage).wait()
            rdA.wait_recv()
            rdA.wait_send()
            cur_ref[stage, pl.ds(0, HT), :] = (
                agA[recv_slot].astype(jnp.float32) * scale)
            if g <= (N_DEV - 1) - S:
                credit_to(creditA)
            if g < N_DEV - 2:
                if g + 1 >= S - 1:
                    pl.semaphore_wait(creditA, 1)
                rdA = ag_send(agA, agA_send, agA_recv,
                              recv_slot, (g + 2) % S)
            rdB.wait_recv()
            rdB.wait_send()
            cur_ref[stage, pl.ds(HT, HT), :] = (
                agB[recv_slot].astype(jnp.float32) * scale)
            if g <= (N_DEV - 1) - S:
                credit_to(creditB)
            if g < N_DEV - 2:
                if g + 1 >= S - 1:
                    pl.semaphore_wait(creditB, 1)
                rdB = ag_send(agB, agB_send, agB_recv,
                              recv_slot, (g + 2) % S)
            c = jnp.mod(p - g, N_DEV)
            st = pltpu.make_async_copy(cur_ref.at[stage],
                                       out_ref.at[pl.ds(c * CH, CH), :],
                                       store_sems.at[stage])
            st.start()
            pending_store[stage] = st
        for st in pending_store.values():
            st.wait()

    return pl.pallas_call(
        body,
        out_shape=jax.ShapeDtypeStruct((M, N), jnp.float32),
        in_specs=[pl.BlockSpec(memory_space=pltpu.VMEM),
                  pl.BlockSpec(memory_space=pltpu.VMEM)],
        out_specs=pl.BlockSpec(memory_space=pl.ANY),
        scratch_shapes=[
            pltpu.VMEM((2, CH, N), jnp.float32),
            pltpu.VMEM((2, HT, N), jnp.bfloat16),
            pltpu.VMEM((2, HT, N), jnp.bfloat16),
            pltpu.VMEM((S, HT, N), jnp.bfloat16),
            pltpu.VMEM((S, HT, N), jnp.bfloat16),
            pltpu.VMEM((S, HT, N), jnp.int8),
            pltpu.VMEM((S, HT, N), jnp.int8),
            pltpu.VMEM((8, 8, 128), jnp.float32),
            pltpu.SemaphoreType.DMA((S,)),
            pltpu.SemaphoreType.DMA((S,)),
            pltpu.SemaphoreType.DMA((S,)),
            pltpu.SemaphoreType.DMA((S,)),
            pltpu.SemaphoreType.DMA((S,)),
            pltpu.SemaphoreType.DMA((S,)),
            pltpu.SemaphoreType.DMA((S,)),
            pltpu.SemaphoreType.DMA((S,)),
            pltpu.SemaphoreType.DMA((5,)),
            pltpu.SemaphoreType.DMA((5,)),
            pltpu.SemaphoreType.DMA((2,)),
            pltpu.SemaphoreType.REGULAR,
            pltpu.SemaphoreType.REGULAR,
        ],
        compiler_params=pltpu.CompilerParams(
            collective_id=0, vmem_limit_bytes=100 * 1024 * 1024),
    )(x, w_mat)
